# baseline (speedup 1.0000x reference)
import functools

import jax
import jax.numpy as jnp
from jax import lax
from jax.experimental import pallas as pl
from jax.experimental.pallas import tpu as pltpu
from jax.experimental.pallas import tpu_sc as plsc

_VOCAB = 100000
_HIDDEN = 768
_PRE_SEQ_LEN = 128
_BATCH = 16
_SEQ = 2048
_PAD_ID = 0
_UNK_ID = 1
_SPECIAL_ID = 99999

_NC = 2
_NS = 16
_L = 16
_NW = _NC * _NS
_B = _BATCH * _SEQ
_BPW = _B // _NW
_C = 32
_NCHUNK = _BPW // _C
_NBUF = 4

_sc_mesh = plsc.VectorSubcoreMesh(
    core_axis_name="c", subcore_axis_name="s",
    num_cores=_NC, num_subcores=_NS,
)


@functools.partial(
    pl.kernel,
    out_type=jax.ShapeDtypeStruct((_B, _HIDDEN), jnp.float32),
    mesh=_sc_mesh,
    scratch_types=[
        pltpu.VMEM((_BPW,), jnp.int32),
        pltpu.VMEM((_NBUF, _C, _HIDDEN), jnp.float32),
        [pltpu.SemaphoreType.DMA] * _NBUF,
        [pltpu.SemaphoreType.DMA] * _NBUF,
    ],
)
def _gather_sc(ids_hbm, table_hbm, out_hbm, idx_v, rows_v, gsem, osem):
    wid = lax.axis_index("s") * _NC + lax.axis_index("c")
    b = wid // 2
    half = wid % 2
    base = wid * _BPW
    pltpu.sync_copy(ids_hbm.at[b, pl.ds(half * _BPW, _BPW)], idx_v)

    @pl.loop(0, _BPW // _L)
    def _remap(j):
        off = j * _L
        v = idx_v[pl.ds(off, _L)]
        idx_v[pl.ds(off, _L)] = jnp.where(v == _SPECIAL_ID, _UNK_ID, v)

    def start_gather(c, buf):
        return pltpu.async_copy(
            table_hbm.at[idx_v.at[pl.ds(c * _C, _C)]], rows_v.at[buf],
            gsem[buf])

    def start_out(c, buf):
        return pltpu.async_copy(
            rows_v.at[buf], out_hbm.at[pl.ds(base + c * _C, _C)], osem[buf])

    def wait_gather(buf):
        pltpu.make_async_copy(
            table_hbm.at[idx_v.at[pl.ds(0, _C)]], rows_v.at[buf],
            gsem[buf]).wait()

    def wait_out(buf):
        pltpu.make_async_copy(
            rows_v.at[buf], out_hbm.at[pl.ds(base, _C)], osem[buf]).wait()

    for bj in range(_NBUF):
        start_gather(bj, bj)

    nrounds = _NCHUNK // _NBUF

    @pl.loop(0, nrounds - 1)
    def _round(r):
        c0 = r * _NBUF
        for bj in range(_NBUF):
            wait_gather(bj)
            start_out(c0 + bj, bj)
        for bj in range(_NBUF):
            wait_out(bj)
            start_gather(c0 + _NBUF + bj, bj)

    for bj in range(_NBUF):
        wait_gather(bj)
        start_out((nrounds - 1) * _NBUF + bj, bj)
    for bj in range(_NBUF):
        wait_out(bj)


def _mlp_mask_body(ids_ref, emb_ref, w1_ref, b1_ref, w2_ref, b2_ref,
                   prompts_ref, mask_ref):
    h = jnp.dot(emb_ref[...], w1_ref[...], preferred_element_type=jnp.float32)
    h = jnp.maximum(h + b1_ref[...], 0.0)
    h = jnp.dot(h, w2_ref[...], preferred_element_type=jnp.float32)
    h = jnp.maximum(h + b2_ref[...], 0.0)
    prompts_ref[...] = jnp.broadcast_to(h[None], (_BATCH, _PRE_SEQ_LEN, _HIDDEN))
    mask_ref[...] = (ids_ref[...] != _PAD_ID).astype(jnp.int8)


_mlp_mask = pl.pallas_call(
    _mlp_mask_body,
    out_shape=(
        jax.ShapeDtypeStruct((_BATCH, _PRE_SEQ_LEN, _HIDDEN), jnp.float32),
        jax.ShapeDtypeStruct((_BATCH, _SEQ), jnp.int8),
    ),
)


def kernel(input_ids, sentences_ids, W_embed, nomal_emb, W1, b1, W2, b2):
    ids32 = input_ids.astype(jnp.int32)
    flat = _gather_sc(ids32, W_embed)
    inputs_embeds = flat.reshape(_BATCH, _SEQ, _HIDDEN)
    prompts, mask8 = _mlp_mask(
        ids32, nomal_emb, W1, b1.reshape(1, _HIDDEN), W2, b2.reshape(1, _HIDDEN)
    )
    return inputs_embeds, prompts, mask8.astype(jnp.bool_)


# speedup vs baseline: 1.0355x; 1.0355x over previous
import functools

import jax
import jax.numpy as jnp
from jax import lax
from jax.experimental import pallas as pl
from jax.experimental.pallas import tpu as pltpu
from jax.experimental.pallas import tpu_sc as plsc

_VOCAB = 100000
_HIDDEN = 768
_PRE_SEQ_LEN = 128
_BATCH = 16
_SEQ = 2048
_PAD_ID = 0
_UNK_ID = 1
_SPECIAL_ID = 99999

_NC = 2
_NS = 16
_L = 16
_NW = _NC * _NS
_B = _BATCH * _SEQ
_BPW = _B // _NW
_C = 32
_NCHUNK = _BPW // _C
_NBUF = 4

_sc_mesh = plsc.VectorSubcoreMesh(
    core_axis_name="c", subcore_axis_name="s",
    num_cores=_NC, num_subcores=_NS,
)


@functools.partial(
    pl.kernel,
    out_type=jax.ShapeDtypeStruct((_B, _HIDDEN), jnp.float32),
    mesh=_sc_mesh,
    scratch_types=[
        pltpu.VMEM((_BPW,), jnp.int32),
        pltpu.VMEM((_NBUF, _C, _HIDDEN), jnp.float32),
        [pltpu.SemaphoreType.DMA] * _NBUF,
        [pltpu.SemaphoreType.DMA] * _NBUF,
    ],
)
def _gather_sc(ids_hbm, table_hbm, out_hbm, idx_v, rows_v, gsem, osem):
    wid = lax.axis_index("s") * _NC + lax.axis_index("c")
    b = wid // 2
    half = wid % 2
    base = wid * _BPW
    pltpu.sync_copy(ids_hbm.at[b, pl.ds(half * _BPW, _BPW)], idx_v)

    def remap(c):
        for j in range(_C // _L):
            off = c * _C + j * _L
            v = idx_v[pl.ds(off, _L)]
            idx_v[pl.ds(off, _L)] = jnp.where(v == _SPECIAL_ID, _UNK_ID, v)

    def start_gather(c, buf):
        return pltpu.async_copy(
            table_hbm.at[idx_v.at[pl.ds(c * _C, _C)]], rows_v.at[buf],
            gsem[buf])

    def start_out(c, buf):
        return pltpu.async_copy(
            rows_v.at[buf], out_hbm.at[pl.ds(base + c * _C, _C)], osem[buf])

    def wait_gather(buf):
        pltpu.make_async_copy(
            table_hbm.at[idx_v.at[pl.ds(0, _C)]], rows_v.at[buf],
            gsem[buf]).wait()

    def wait_out(buf):
        pltpu.make_async_copy(
            rows_v.at[buf], out_hbm.at[pl.ds(base, _C)], osem[buf]).wait()

    out_pending = [False] * _NBUF
    remap(0)
    start_gather(0, 0)
    remap(1)
    start_gather(1, 1)
    for c in range(_NCHUNK):
        wait_gather(c % _NBUF)
        nc = c + 2
        if nc < _NCHUNK:
            nbuf = nc % _NBUF
            if out_pending[nbuf]:
                wait_out(nbuf)
                out_pending[nbuf] = False
            remap(nc)
            start_gather(nc, nbuf)
        start_out(c, c % _NBUF)
        out_pending[c % _NBUF] = True
    for bj in range(_NBUF):
        if out_pending[bj]:
            wait_out(bj)


def _mlp_mask_body(ids_ref, emb_ref, w1_ref, b1_ref, w2_ref, b2_ref,
                   prompts_ref, mask_ref):
    h = jnp.dot(emb_ref[...], w1_ref[...], preferred_element_type=jnp.float32)
    h = jnp.maximum(h + b1_ref[...], 0.0)
    h = jnp.dot(h, w2_ref[...], preferred_element_type=jnp.float32)
    h = jnp.maximum(h + b2_ref[...], 0.0)
    prompts_ref[...] = jnp.broadcast_to(h[None], (_BATCH, _PRE_SEQ_LEN, _HIDDEN))
    mask_ref[...] = (ids_ref[...] != _PAD_ID).astype(jnp.int8)


_mlp_mask = pl.pallas_call(
    _mlp_mask_body,
    out_shape=(
        jax.ShapeDtypeStruct((_BATCH, _PRE_SEQ_LEN, _HIDDEN), jnp.float32),
        jax.ShapeDtypeStruct((_BATCH, _SEQ), jnp.int8),
    ),
)


def kernel(input_ids, sentences_ids, W_embed, nomal_emb, W1, b1, W2, b2):
    ids32 = input_ids.astype(jnp.int32)
    flat = _gather_sc(ids32, W_embed)
    inputs_embeds = flat.reshape(_BATCH, _SEQ, _HIDDEN)
    prompts, mask8 = _mlp_mask(
        ids32, nomal_emb, W1, b1.reshape(1, _HIDDEN), W2, b2.reshape(1, _HIDDEN)
    )
    return inputs_embeds, prompts, mask8.astype(jnp.bool_)


# speedup vs baseline: 1.0421x; 1.0064x over previous
import functools

import jax
import jax.numpy as jnp
from jax import lax
from jax.experimental import pallas as pl
from jax.experimental.pallas import tpu as pltpu
from jax.experimental.pallas import tpu_sc as plsc

_VOCAB = 100000
_HIDDEN = 768
_PRE_SEQ_LEN = 128
_BATCH = 16
_SEQ = 2048
_PAD_ID = 0
_UNK_ID = 1
_SPECIAL_ID = 99999

_NC = 2
_NS = 16
_L = 16
_NW = _NC * _NS
_B = _BATCH * _SEQ
_BPW = _B // _NW
_C = 64
_NCHUNK = _BPW // _C
_NBUF = 2

_sc_mesh = plsc.VectorSubcoreMesh(
    core_axis_name="c", subcore_axis_name="s",
    num_cores=_NC, num_subcores=_NS,
)


@functools.partial(
    pl.kernel,
    out_type=jax.ShapeDtypeStruct((_B, _HIDDEN), jnp.float32),
    mesh=_sc_mesh,
    scratch_types=[
        pltpu.VMEM((_BPW,), jnp.int32),
        pltpu.VMEM((_NBUF, _C, _HIDDEN), jnp.float32),
        [pltpu.SemaphoreType.DMA] * _NBUF,
        [pltpu.SemaphoreType.DMA] * _NBUF,
    ],
)
def _gather_sc(ids_hbm, table_hbm, out_hbm, idx_v, rows_v, gsem, osem):
    wid = lax.axis_index("s") * _NC + lax.axis_index("c")
    b = wid // 2
    half = wid % 2
    base = wid * _BPW
    pltpu.sync_copy(ids_hbm.at[b, pl.ds(half * _BPW, _BPW)], idx_v)

    def remap(c):
        for j in range(_C // _L):
            off = c * _C + j * _L
            v = idx_v[pl.ds(off, _L)]
            idx_v[pl.ds(off, _L)] = jnp.where(v == _SPECIAL_ID, _UNK_ID, v)

    def start_gather(c, buf):
        return pltpu.async_copy(
            table_hbm.at[idx_v.at[pl.ds(c * _C, _C)]], rows_v.at[buf],
            gsem[buf])

    def start_out(c, buf):
        return pltpu.async_copy(
            rows_v.at[buf], out_hbm.at[pl.ds(base + c * _C, _C)], osem[buf])

    def wait_gather(buf):
        pltpu.make_async_copy(
            table_hbm.at[idx_v.at[pl.ds(0, _C)]], rows_v.at[buf],
            gsem[buf]).wait()

    def wait_out(buf):
        pltpu.make_async_copy(
            rows_v.at[buf], out_hbm.at[pl.ds(base, _C)], osem[buf]).wait()

    out_pending = [False] * _NBUF
    remap(0)
    start_gather(0, 0)
    remap(1)
    start_gather(1, 1)
    for c in range(_NCHUNK):
        wait_gather(c % _NBUF)
        nc = c + 2
        if nc < _NCHUNK:
            nbuf = nc % _NBUF
            if out_pending[nbuf]:
                wait_out(nbuf)
                out_pending[nbuf] = False
            remap(nc)
            start_gather(nc, nbuf)
        start_out(c, c % _NBUF)
        out_pending[c % _NBUF] = True
    for bj in range(_NBUF):
        if out_pending[bj]:
            wait_out(bj)


def _mlp_mask_body(ids_ref, emb_ref, w1_ref, b1_ref, w2_ref, b2_ref,
                   prompts_ref, mask_ref):
    h = jnp.dot(emb_ref[...], w1_ref[...], preferred_element_type=jnp.float32)
    h = jnp.maximum(h + b1_ref[...], 0.0)
    h = jnp.dot(h, w2_ref[...], preferred_element_type=jnp.float32)
    h = jnp.maximum(h + b2_ref[...], 0.0)
    prompts_ref[...] = jnp.broadcast_to(h[None], (_BATCH, _PRE_SEQ_LEN, _HIDDEN))
    mask_ref[...] = (ids_ref[...] != _PAD_ID).astype(jnp.int8)


_mlp_mask = pl.pallas_call(
    _mlp_mask_body,
    out_shape=(
        jax.ShapeDtypeStruct((_BATCH, _PRE_SEQ_LEN, _HIDDEN), jnp.float32),
        jax.ShapeDtypeStruct((_BATCH, _SEQ), jnp.int8),
    ),
)


def kernel(input_ids, sentences_ids, W_embed, nomal_emb, W1, b1, W2, b2):
    ids32 = input_ids.astype(jnp.int32)
    flat = _gather_sc(ids32, W_embed)
    inputs_embeds = flat.reshape(_BATCH, _SEQ, _HIDDEN)
    prompts, mask8 = _mlp_mask(
        ids32, nomal_emb, W1, b1.reshape(1, _HIDDEN), W2, b2.reshape(1, _HIDDEN)
    )
    return inputs_embeds, prompts, mask8.astype(jnp.bool_)
